# baseline (device time: 41734 ns/iter reference)
import jax
import jax.numpy as jnp
from jax import lax
from jax.experimental import pallas as pl
from jax.experimental.pallas import tpu as pltpu

N_DEV = 4


def kernel(x, Win0, Wout0, Win1, Wout1, Win2, Wout2):
    m_per, d = x.shape
    B = N_DEV * m_per

    def body(x_ref, win0, wout0, win1, wout1, win2, wout2,
             out_ref, xfull, acc_a, acc_b, send_sems, recv_sems):
        my_pos = lax.axis_index("i")
        pa = my_pos ^ 1
        pb = (N_DEV - 1) - my_pos

        barrier = pltpu.get_barrier_semaphore()
        for nbr in (pa, pb):
            pl.semaphore_signal(barrier, inc=1, device_id=(nbr,),
                                device_id_type=pl.DeviceIdType.MESH)
        pl.semaphore_wait(barrier, 2)

        def exchange(src, dst, sem_idx, partner):
            rdma = pltpu.make_async_remote_copy(
                src_ref=src, dst_ref=dst,
                send_sem=send_sems.at[sem_idx],
                recv_sem=recv_sems.at[sem_idx],
                device_id=(partner,),
                device_id_type=pl.DeviceIdType.MESH,
            )
            rdma.start()
            rdma.wait()

        row0 = my_pos * m_per
        xfull[pl.ds(row0, m_per), :] = x_ref[...]
        exchange(xfull.at[pl.ds(row0, m_per), :],
                 xfull.at[pl.ds(row0, m_per), :], 0, pa)
        pair_row0 = (my_pos // 2) * (2 * m_per)
        exchange(xfull.at[pl.ds(pair_row0, 2 * m_per), :],
                 xfull.at[pl.ds(pair_row0, 2 * m_per), :], 1, pb)

        xv = xfull[...]
        layers = ((win0, wout0), (win1, wout1), (win2, wout2))
        for l, (win, wout) in enumerate(layers):
            h = jnp.maximum(
                jnp.dot(xv, win[...], preferred_element_type=jnp.float32),
                0.0)
            p = jnp.dot(h, wout[...], preferred_element_type=jnp.float32)
            acc_a[l, 0] = p
            exchange(acc_a.at[l, 0], acc_a.at[l, 1], 2 + 2 * l, pa)
            acc_b[l, 0] = acc_a[l, 0] + acc_a[l, 1]
            exchange(acc_b.at[l, 0], acc_b.at[l, 1], 3 + 2 * l, pb)
            xv = acc_b[l, 0] + acc_b[l, 1]

        out_ref[...] = xv

    return pl.pallas_call(
        body,
        out_shape=jax.ShapeDtypeStruct((B, d), jnp.float32),
        in_specs=[pl.BlockSpec(memory_space=pltpu.VMEM)] * 7,
        out_specs=pl.BlockSpec(memory_space=pltpu.VMEM),
        scratch_shapes=[
            pltpu.VMEM((B, d), jnp.float32),
            pltpu.VMEM((3, 2, B, d), jnp.float32),
            pltpu.VMEM((3, 2, B, d), jnp.float32),
            pltpu.SemaphoreType.DMA((8,)),
            pltpu.SemaphoreType.DMA((8,)),
        ],
        compiler_params=pltpu.CompilerParams(collective_id=0),
    )(x, Win0, Wout0, Win1, Wout1, Win2, Wout2)


# device time: 35202 ns/iter; 1.1856x vs baseline; 1.1856x over previous
import jax
import jax.numpy as jnp
from jax import lax
from jax.experimental import pallas as pl
from jax.experimental.pallas import tpu as pltpu

N_DEV = 4


def kernel(x, Win0, Wout0, Win1, Wout1, Win2, Wout2):
    m_per, d = x.shape
    B = N_DEV * m_per
    H = B // 2

    def body(x_ref, win0, wout0, win1, wout1, win2, wout2,
             out_ref, xfull, pbuf, r1, sbuf, r2, send_sems, recv_sems):
        my_pos = lax.axis_index("i")
        pa = my_pos ^ 1
        pb = (N_DEV - 1) - my_pos

        barrier = pltpu.get_barrier_semaphore()
        for nbr in (pa, pb):
            pl.semaphore_signal(barrier, inc=1, device_id=(nbr,),
                                device_id_type=pl.DeviceIdType.MESH)
        pl.semaphore_wait(barrier, 2)

        pending_sends = []

        def start(src, dst, sem_idx, partner):
            rdma = pltpu.make_async_remote_copy(
                src_ref=src, dst_ref=dst,
                send_sem=send_sems.at[sem_idx],
                recv_sem=recv_sems.at[sem_idx],
                device_id=(partner,),
                device_id_type=pl.DeviceIdType.MESH,
            )
            rdma.start()
            pending_sends.append(rdma)
            return rdma

        row0 = my_pos * m_per
        xfull[pl.ds(row0, m_per), :] = x_ref[...]
        ag1 = start(xfull.at[pl.ds(row0, m_per), :],
                    xfull.at[pl.ds(row0, m_per), :], 0, pa)
        ag1.wait_recv()
        pair_row0 = (my_pos // 2) * (2 * m_per)
        ag2 = start(xfull.at[pl.ds(pair_row0, 2 * m_per), :],
                    xfull.at[pl.ds(pair_row0, 2 * m_per), :], 1, pb)
        ag2.wait_recv()

        xtop = xfull[pl.ds(0, H), :]
        xbot = xfull[pl.ds(H, H), :]

        layers = ((win0, wout0), (win1, wout1), (win2, wout2))
        pend_bot = None
        for l, (win, wout) in enumerate(layers):
            s = 2 + 4 * l
            ht = jnp.maximum(
                jnp.dot(xtop, win[...], preferred_element_type=jnp.float32), 0.0)
            pt = jnp.dot(ht, wout[...], preferred_element_type=jnp.float32)
            pbuf[l, 0] = pt
            a1t = start(pbuf.at[l, 0], r1.at[l, 0], s, pa)
            if pend_bot is not None:
                desc, pl_ = pend_bot
                desc.wait_recv()
                xbot = sbuf[pl_, 1] + r2[pl_, 1]
            hb = jnp.maximum(
                jnp.dot(xbot, win[...], preferred_element_type=jnp.float32), 0.0)
            pb_ = jnp.dot(hb, wout[...], preferred_element_type=jnp.float32)
            pbuf[l, 1] = pb_
            a1b = start(pbuf.at[l, 1], r1.at[l, 1], s + 1, pb)
            a1t.wait_recv()
            sbuf[l, 0] = pt + r1[l, 0]
            a2t = start(sbuf.at[l, 0], r2.at[l, 0], s + 2, pb)
            a1b.wait_recv()
            sbuf[l, 1] = pb_ + r1[l, 1]
            a2b = start(sbuf.at[l, 1], r2.at[l, 1], s + 3, pa)
            a2t.wait_recv()
            xtop = sbuf[l, 0] + r2[l, 0]
            pend_bot = (a2b, l)

        desc, pl_ = pend_bot
        desc.wait_recv()
        xbot = sbuf[pl_, 1] + r2[pl_, 1]

        out_ref[pl.ds(0, H), :] = xtop
        out_ref[pl.ds(H, H), :] = xbot

        for rdma in pending_sends:
            rdma.wait_send()

    return pl.pallas_call(
        body,
        out_shape=jax.ShapeDtypeStruct((B, d), jnp.float32),
        in_specs=[pl.BlockSpec(memory_space=pltpu.VMEM)] * 7,
        out_specs=pl.BlockSpec(memory_space=pltpu.VMEM),
        scratch_shapes=[
            pltpu.VMEM((B, d), jnp.float32),
            pltpu.VMEM((3, 2, H, d), jnp.float32),
            pltpu.VMEM((3, 2, H, d), jnp.float32),
            pltpu.VMEM((3, 2, H, d), jnp.float32),
            pltpu.VMEM((3, 2, H, d), jnp.float32),
            pltpu.SemaphoreType.DMA((14,)),
            pltpu.SemaphoreType.DMA((14,)),
        ],
        compiler_params=pltpu.CompilerParams(collective_id=0),
    )(x, Win0, Wout0, Win1, Wout1, Win2, Wout2)


# device time: 29933 ns/iter; 1.3942x vs baseline; 1.1760x over previous
import jax
import jax.numpy as jnp
from jax import lax
from jax.experimental import pallas as pl
from jax.experimental.pallas import tpu as pltpu

N_DEV = 4


def kernel(x, Win0, Wout0, Win1, Wout1, Win2, Wout2):
    m_per, d = x.shape
    B = N_DEV * m_per
    H = B // 2

    def body(x_ref, win0, wout0, win1, wout1, win2, wout2,
             out_ref, xfull, pbuf, r1, sbuf, r2, send_sems, recv_sems):
        my_pos = lax.axis_index("i")
        pa = my_pos ^ 1
        pb = (N_DEV - 1) - my_pos

        barrier = pltpu.get_barrier_semaphore()
        for nbr in (pa, pb):
            pl.semaphore_signal(barrier, inc=1, device_id=(nbr,),
                                device_id_type=pl.DeviceIdType.MESH)
        pl.semaphore_wait(barrier, 2)

        pending_sends = []

        def start(src, dst, sem_idx, partner):
            rdma = pltpu.make_async_remote_copy(
                src_ref=src, dst_ref=dst,
                send_sem=send_sems.at[sem_idx],
                recv_sem=recv_sems.at[sem_idx],
                device_id=(partner,),
                device_id_type=pl.DeviceIdType.MESH,
            )
            rdma.start()
            pending_sends.append(rdma)
            return rdma

        row0 = my_pos * m_per
        xfull[pl.ds(row0, m_per), :] = x_ref[...].astype(jnp.bfloat16)
        ag1 = start(xfull.at[pl.ds(row0, m_per), :],
                    xfull.at[pl.ds(row0, m_per), :], 0, pa)
        ag1.wait_recv()
        pair_row0 = (my_pos // 2) * (2 * m_per)
        ag2 = start(xfull.at[pl.ds(pair_row0, 2 * m_per), :],
                    xfull.at[pl.ds(pair_row0, 2 * m_per), :], 1, pb)
        ag2.wait_recv()

        xtop = xfull[pl.ds(0, H), :].astype(jnp.float32)
        xbot = xfull[pl.ds(H, H), :].astype(jnp.float32)

        layers = ((win0, wout0), (win1, wout1), (win2, wout2))
        pend_bot = None
        for l, (win, wout) in enumerate(layers):
            s = 2 + 4 * l
            ht = jnp.maximum(
                jnp.dot(xtop, win[...], preferred_element_type=jnp.float32), 0.0)
            pt = jnp.dot(ht, wout[...], preferred_element_type=jnp.float32)
            pbuf[l, 0] = pt.astype(jnp.bfloat16)
            a1t = start(pbuf.at[l, 0], r1.at[l, 0], s, pa)
            if pend_bot is not None:
                desc, pl_ = pend_bot
                desc.wait_recv()
                xbot = (sbuf[pl_, 1].astype(jnp.float32)
                        + r2[pl_, 1].astype(jnp.float32))
            hb = jnp.maximum(
                jnp.dot(xbot, win[...], preferred_element_type=jnp.float32), 0.0)
            pb_ = jnp.dot(hb, wout[...], preferred_element_type=jnp.float32)
            pbuf[l, 1] = pb_.astype(jnp.bfloat16)
            a1b = start(pbuf.at[l, 1], r1.at[l, 1], s + 1, pb)
            a1t.wait_recv()
            sbuf[l, 0] = (pt + r1[l, 0].astype(jnp.float32)).astype(jnp.bfloat16)
            a2t = start(sbuf.at[l, 0], r2.at[l, 0], s + 2, pb)
            a1b.wait_recv()
            sbuf[l, 1] = (pb_ + r1[l, 1].astype(jnp.float32)).astype(jnp.bfloat16)
            a2b = start(sbuf.at[l, 1], r2.at[l, 1], s + 3, pa)
            a2t.wait_recv()
            xtop = (sbuf[l, 0].astype(jnp.float32)
                    + r2[l, 0].astype(jnp.float32))
            pend_bot = (a2b, l)

        desc, pl_ = pend_bot
        desc.wait_recv()
        xbot = (sbuf[pl_, 1].astype(jnp.float32)
                + r2[pl_, 1].astype(jnp.float32))

        out_ref[pl.ds(0, H), :] = xtop
        out_ref[pl.ds(H, H), :] = xbot

        for rdma in pending_sends:
            rdma.wait_send()

    return pl.pallas_call(
        body,
        out_shape=jax.ShapeDtypeStruct((B, d), jnp.float32),
        in_specs=[pl.BlockSpec(memory_space=pltpu.VMEM)] * 7,
        out_specs=pl.BlockSpec(memory_space=pltpu.VMEM),
        scratch_shapes=[
            pltpu.VMEM((B, d), jnp.bfloat16),
            pltpu.VMEM((3, 2, H, d), jnp.bfloat16),
            pltpu.VMEM((3, 2, H, d), jnp.bfloat16),
            pltpu.VMEM((3, 2, H, d), jnp.bfloat16),
            pltpu.VMEM((3, 2, H, d), jnp.bfloat16),
            pltpu.SemaphoreType.DMA((14,)),
            pltpu.SemaphoreType.DMA((14,)),
        ],
        compiler_params=pltpu.CompilerParams(collective_id=0),
    )(x, Win0, Wout0, Win1, Wout1, Win2, Wout2)


# device time: 28463 ns/iter; 1.4663x vs baseline; 1.0516x over previous
import jax
import jax.numpy as jnp
from jax import lax
from jax.experimental import pallas as pl
from jax.experimental.pallas import tpu as pltpu

N_DEV = 4


def kernel(x, Win0, Wout0, Win1, Wout1, Win2, Wout2):
    m_per, d = x.shape
    B = N_DEV * m_per
    H = B // 2

    def body(x_ref, win0, wout0, win1, wout1, win2, wout2,
             out_ref, xfull, pbuf, r1, sbuf, r2, send_sems, recv_sems):
        my_pos = lax.axis_index("i")
        pa = my_pos ^ 1
        pb = (N_DEV - 1) - my_pos

        barrier = pltpu.get_barrier_semaphore()
        for nbr in (pa, pb):
            pl.semaphore_signal(barrier, inc=1, device_id=(nbr,),
                                device_id_type=pl.DeviceIdType.MESH)
        pl.semaphore_wait(barrier, 2)

        pending_sends = []

        def start(src, dst, sem_idx, partner):
            rdma = pltpu.make_async_remote_copy(
                src_ref=src, dst_ref=dst,
                send_sem=send_sems.at[sem_idx],
                recv_sem=recv_sems.at[sem_idx],
                device_id=(partner,),
                device_id_type=pl.DeviceIdType.MESH,
            )
            rdma.start()
            pending_sends.append(rdma)
            return rdma

        pd = my_pos ^ 2
        row0 = my_pos * m_per
        xfull[pl.ds(row0, m_per), :] = x_ref[...].astype(jnp.bfloat16)
        my_rows = xfull.at[pl.ds(row0, m_per), :]
        for idx, peer in ((0, pa), (1, pb), (2, pd)):
            start(my_rows, my_rows, idx, peer)
        for idx, peer in ((0, pa), (1, pb), (2, pd)):
            recv = pltpu.make_async_remote_copy(
                src_ref=my_rows,
                dst_ref=xfull.at[pl.ds(peer * m_per, m_per), :],
                send_sem=send_sems.at[idx],
                recv_sem=recv_sems.at[idx],
                device_id=(peer,),
                device_id_type=pl.DeviceIdType.MESH,
            )
            recv.wait_recv()

        xtop = xfull[pl.ds(0, H), :].astype(jnp.float32)
        xbot = xfull[pl.ds(H, H), :].astype(jnp.float32)

        layers = ((win0, wout0), (win1, wout1), (win2, wout2))
        pend_bot = None
        for l, (win, wout) in enumerate(layers):
            s = 3 + 4 * l
            ht = jnp.maximum(
                jnp.dot(xtop, win[...], preferred_element_type=jnp.float32), 0.0)
            pt = jnp.dot(ht, wout[...], preferred_element_type=jnp.float32)
            pbuf[l, 0] = pt.astype(jnp.bfloat16)
            a1t = start(pbuf.at[l, 0], r1.at[l, 0], s, pa)
            if pend_bot is not None:
                desc, pl_ = pend_bot
                desc.wait_recv()
                xbot = (sbuf[pl_, 1].astype(jnp.float32)
                        + r2[pl_, 1].astype(jnp.float32))
            hb = jnp.maximum(
                jnp.dot(xbot, win[...], preferred_element_type=jnp.float32), 0.0)
            pb_ = jnp.dot(hb, wout[...], preferred_element_type=jnp.float32)
            pbuf[l, 1] = pb_.astype(jnp.bfloat16)
            a1b = start(pbuf.at[l, 1], r1.at[l, 1], s + 1, pb)
            a1t.wait_recv()
            sbuf[l, 0] = (pt + r1[l, 0].astype(jnp.float32)).astype(jnp.bfloat16)
            a2t = start(sbuf.at[l, 0], r2.at[l, 0], s + 2, pb)
            a1b.wait_recv()
            sbuf[l, 1] = (pb_ + r1[l, 1].astype(jnp.float32)).astype(jnp.bfloat16)
            a2b = start(sbuf.at[l, 1], r2.at[l, 1], s + 3, pa)
            a2t.wait_recv()
            xtop = (sbuf[l, 0].astype(jnp.float32)
                    + r2[l, 0].astype(jnp.float32))
            pend_bot = (a2b, l)

        desc, pl_ = pend_bot
        desc.wait_recv()
        xbot = (sbuf[pl_, 1].astype(jnp.float32)
                + r2[pl_, 1].astype(jnp.float32))

        out_ref[pl.ds(0, H), :] = xtop
        out_ref[pl.ds(H, H), :] = xbot

        for rdma in pending_sends:
            rdma.wait_send()

    return pl.pallas_call(
        body,
        out_shape=jax.ShapeDtypeStruct((B, d), jnp.float32),
        in_specs=[pl.BlockSpec(memory_space=pltpu.VMEM)] * 7,
        out_specs=pl.BlockSpec(memory_space=pltpu.VMEM),
        scratch_shapes=[
            pltpu.VMEM((B, d), jnp.bfloat16),
            pltpu.VMEM((3, 2, H, d), jnp.bfloat16),
            pltpu.VMEM((3, 2, H, d), jnp.bfloat16),
            pltpu.VMEM((3, 2, H, d), jnp.bfloat16),
            pltpu.VMEM((3, 2, H, d), jnp.bfloat16),
            pltpu.SemaphoreType.DMA((15,)),
            pltpu.SemaphoreType.DMA((15,)),
        ],
        compiler_params=pltpu.CompilerParams(collective_id=0),
    )(x, Win0, Wout0, Win1, Wout1, Win2, Wout2)


# device time: 27760 ns/iter; 1.5034x vs baseline; 1.0253x over previous
import jax
import jax.numpy as jnp
from jax import lax
from jax.experimental import pallas as pl
from jax.experimental.pallas import tpu as pltpu

N_DEV = 4


def kernel(x, Win0, Wout0, Win1, Wout1, Win2, Wout2):
    m_per, d = x.shape
    B = N_DEV * m_per
    H = B // 2
    Q = H // 2

    f32 = jnp.float32
    bf16 = jnp.bfloat16

    def body(x_ref, win0, wout0, win1, wout1, win2, wout2,
             out_ref, xfull, pbuf, r1, sbuf, r2, send_sems, recv_sems):
        my_pos = lax.axis_index("i")
        pa = my_pos ^ 1
        pb = (N_DEV - 1) - my_pos
        pd = my_pos ^ 2

        barrier = pltpu.get_barrier_semaphore()
        for nbr in (pa, pb):
            pl.semaphore_signal(barrier, inc=1, device_id=(nbr,),
                                device_id_type=pl.DeviceIdType.MESH)
        pl.semaphore_wait(barrier, 2)

        pending_sends = []

        def start(src, dst, sem_idx, partner):
            rdma = pltpu.make_async_remote_copy(
                src_ref=src, dst_ref=dst,
                send_sem=send_sems.at[sem_idx],
                recv_sem=recv_sems.at[sem_idx],
                device_id=(partner,),
                device_id_type=pl.DeviceIdType.MESH,
            )
            rdma.start()
            pending_sends.append(rdma)
            return rdma

        row0 = my_pos * m_per
        xfull[pl.ds(row0, m_per), :] = x_ref[...].astype(bf16)
        my_rows = xfull.at[pl.ds(row0, m_per), :]
        for idx, peer in ((0, pa), (1, pb), (2, pd)):
            start(my_rows, my_rows, idx, peer)
        for idx, peer in ((0, pa), (1, pb), (2, pd)):
            recv = pltpu.make_async_remote_copy(
                src_ref=my_rows,
                dst_ref=xfull.at[pl.ds(peer * m_per, m_per), :],
                send_sem=send_sems.at[idx],
                recv_sem=recv_sems.at[idx],
                device_id=(peer,),
                device_id_type=pl.DeviceIdType.MESH,
            )
            recv.wait_recv()

        xtop = xfull[pl.ds(0, H), :]
        xbot = xfull[pl.ds(H, H), :]

        def partial(xh, win_bf, wout_bf):
            h = jnp.maximum(
                jnp.dot(xh, win_bf, preferred_element_type=f32), 0.0)
            return jnp.dot(h.astype(bf16), wout_bf,
                           preferred_element_type=f32)

        layers = ((win0, wout0), (win1, wout1), (win2, wout2))
        pend_bot = None
        for l, (win, wout) in enumerate(layers):
            s = 3 + 8 * l
            win_bf = win[...].astype(bf16)
            wout_bf = wout[...].astype(bf16)
            pt = partial(xtop, win_bf, wout_bf)
            pbuf[l, 0] = pt.astype(bf16)
            a_t = [start(pbuf.at[l, 0, pl.ds(q * Q, Q)],
                         r1.at[l, 0, pl.ds(q * Q, Q)], s + q, pa)
                   for q in (0, 1)]
            if pend_bot is not None:
                cb0, cb1, lp = pend_bot
                cb0.wait_recv()
                cb1.wait_recv()
                xbot = (sbuf[lp, 1].astype(f32)
                        + r2[lp, 1].astype(f32)).astype(bf16)
            pbv = partial(xbot, win_bf, wout_bf)
            pbuf[l, 1] = pbv.astype(bf16)
            a_b = [start(pbuf.at[l, 1, pl.ds(q * Q, Q)],
                         r1.at[l, 1, pl.ds(q * Q, Q)], s + 2 + q, pb)
                   for q in (0, 1)]
            c_t = []
            for q in (0, 1):
                a_t[q].wait_recv()
                sbuf[l, 0, pl.ds(q * Q, Q)] = (
                    pt[q * Q:(q + 1) * Q]
                    + r1[l, 0, pl.ds(q * Q, Q)].astype(f32)).astype(bf16)
                c_t.append(start(sbuf.at[l, 0, pl.ds(q * Q, Q)],
                                 r2.at[l, 0, pl.ds(q * Q, Q)], s + 4 + q, pb))
            c_b = []
            for q in (0, 1):
                a_b[q].wait_recv()
                sbuf[l, 1, pl.ds(q * Q, Q)] = (
                    pbv[q * Q:(q + 1) * Q]
                    + r1[l, 1, pl.ds(q * Q, Q)].astype(f32)).astype(bf16)
                c_b.append(start(sbuf.at[l, 1, pl.ds(q * Q, Q)],
                                 r2.at[l, 1, pl.ds(q * Q, Q)], s + 6 + q, pa))
            c_t[0].wait_recv()
            c_t[1].wait_recv()
            if l < 2:
                xtop = (sbuf[l, 0].astype(f32)
                        + r2[l, 0].astype(f32)).astype(bf16)
            pend_bot = (c_b[0], c_b[1], l)

        out_ref[pl.ds(0, H), :] = sbuf[2, 0].astype(f32) + r2[2, 0].astype(f32)
        cb0, cb1, lp = pend_bot
        cb0.wait_recv()
        cb1.wait_recv()
        out_ref[pl.ds(H, H), :] = (sbuf[lp, 1].astype(f32)
                                   + r2[lp, 1].astype(f32))

        for rdma in pending_sends:
            rdma.wait_send()

    return pl.pallas_call(
        body,
        out_shape=jax.ShapeDtypeStruct((B, d), jnp.float32),
        in_specs=[pl.BlockSpec(memory_space=pltpu.VMEM)] * 7,
        out_specs=pl.BlockSpec(memory_space=pltpu.VMEM),
        scratch_shapes=[
            pltpu.VMEM((B, d), bf16),
            pltpu.VMEM((3, 2, H, d), bf16),
            pltpu.VMEM((3, 2, H, d), bf16),
            pltpu.VMEM((3, 2, H, d), bf16),
            pltpu.VMEM((3, 2, H, d), bf16),
            pltpu.SemaphoreType.DMA((27,)),
            pltpu.SemaphoreType.DMA((27,)),
        ],
        compiler_params=pltpu.CompilerParams(collective_id=0),
    )(x, Win0, Wout0, Win1, Wout1, Win2, Wout2)
